# Initial kernel scaffold; baseline (speedup 1.0000x reference)
import jax
import jax.numpy as jnp
from jax.experimental import pallas as pl


def kernel(x, edge_index, edge_attr, We, be, Wn, bn, W1, b1, W2, b2):
    raise NotImplementedError("write your pallas kernel here")


# speedup vs baseline: 10.4621x; 10.4621x over previous
import functools

import jax
import jax.numpy as jnp
from jax import lax
from jax.experimental import pallas as pl
from jax.experimental.pallas import tpu as pltpu
from jax.experimental.pallas import tpu_sc as plsc

N = 10000
E = 320000
D_NODE = 128
D_EDGE = 16
NODE_OUT = 16
EDGE_OUT = 8
HIDDEN = 10
NUM_CLASSES = 2

NC, NS, L = 2, 16, 16
NW = NC * NS
ROW = 128
NROWS = E // ROW
RW = 80
NROWS_PAD = RW * NW
NSLICE = 632
N_PAD = NSLICE * NS
PAD_NODE = N

F1 = 32
F2 = 16

_mesh = plsc.VectorSubcoreMesh(core_axis_name="c", subcore_axis_name="s",
                               num_cores=NC, num_subcores=NS)
_sc_params = pltpu.CompilerParams(use_tc_tiling_on_sc=False)


def _zero_fill(zbuf, f):
    def body(r, _):
        for j in range(f // L):
            zbuf[r, pl.ds(j * L, L)] = jnp.zeros((L,), jnp.float32)
        return 0
    lax.fori_loop(0, zbuf.shape[0], body, 0)


def _sc_edge_body(attr3d, src3d, dst3d, aggp, degp,
                  ev, sbuf, dbuf, onesd, oness, zbuf, agg_s, deg_s, sem0, sem1):
    c = lax.axis_index("c")
    s = lax.axis_index("s")
    wid = s * NC + c
    _zero_fill(zbuf, 16)

    def ofill(r, _):
        onesd[r, :] = jnp.where(lax.iota(jnp.int32, L) == 0, 1.0, 0.0)
        oness[r, :] = jnp.where(lax.iota(jnp.int32, L) == 1, 1.0, 0.0)
        return 0
    lax.fori_loop(0, ROW, ofill, 0)

    base = s * NSLICE
    pltpu.sync_copy(zbuf, agg_s.at[pl.ds(base, NSLICE)])
    pltpu.sync_copy(zbuf, deg_s.at[pl.ds(base, NSLICE)])

    start = wid * RW
    pltpu.sync_copy(src3d.at[pl.ds(start, RW)], sbuf)
    pltpu.sync_copy(dst3d.at[pl.ds(start, RW)], dbuf)
    pltpu.sync_copy(attr3d.at[0], ev.at[0])
    pltpu.sync_copy(attr3d.at[0], ev.at[1])
    plsc.subcore_barrier()

    def pair(p, _):
        j0 = 2 * p
        j1 = j0 + 1
        r0 = start + j0
        r1 = start + j1

        @pl.when(r0 < NROWS)
        def _():
            pltpu.async_copy(attr3d.at[r0], ev.at[0], sem0)

        @pl.when(r1 < NROWS)
        def _():
            pltpu.async_copy(attr3d.at[r1], ev.at[1], sem1)

        @pl.when(r0 < NROWS)
        def _():
            pltpu.make_async_copy(attr3d.at[r0], ev.at[0], sem0).wait()
        pltpu.sync_copy(ev.at[0], agg_s.at[dbuf.at[j0, 0]], add=True)
        pltpu.sync_copy(onesd, deg_s.at[dbuf.at[j0, 0]], add=True)
        pltpu.sync_copy(oness, deg_s.at[sbuf.at[j0, 0]], add=True)

        @pl.when(r1 < NROWS)
        def _():
            pltpu.make_async_copy(attr3d.at[r1], ev.at[1], sem1).wait()
        pltpu.sync_copy(ev.at[1], agg_s.at[dbuf.at[j1, 0]], add=True)
        pltpu.sync_copy(onesd, deg_s.at[dbuf.at[j1, 0]], add=True)
        pltpu.sync_copy(oness, deg_s.at[sbuf.at[j1, 0]], add=True)
        return 0
    lax.fori_loop(0, RW // 2, pair, 0)
    plsc.subcore_barrier()
    pltpu.sync_copy(agg_s.at[pl.ds(base, NSLICE)], aggp.at[c, pl.ds(base, NSLICE)])
    pltpu.sync_copy(deg_s.at[pl.ds(base, NSLICE)], degp.at[c, pl.ds(base, NSLICE)])


_sc_edge = pl.kernel(
    _sc_edge_body,
    out_type=[pltpu.HBM((NC, N_PAD, 16), jnp.float32),
              pltpu.HBM((NC, N_PAD, 16), jnp.float32)],
    mesh=_mesh,
    compiler_params=_sc_params,
    scratch_types=[
        pltpu.VMEM((2, ROW, 16), jnp.float32),
        pltpu.VMEM((RW, 1, ROW), jnp.int32),
        pltpu.VMEM((RW, 1, ROW), jnp.int32),
        pltpu.VMEM((ROW, 16), jnp.float32),
        pltpu.VMEM((ROW, 16), jnp.float32),
        pltpu.VMEM((NSLICE, 16), jnp.float32),
        pltpu.VMEM_SHARED((N_PAD, 16), jnp.float32),
        pltpu.VMEM_SHARED((N_PAD, 16), jnp.float32),
        pltpu.SemaphoreType.DMA,
        pltpu.SemaphoreType.DMA,
    ],
)


def _sc_conv_body(f, hs, src3d, dst3d, aggp,
                  sbuf, dbuf, rows, zbuf, acc_s, sem0, sem1):
    c = lax.axis_index("c")
    s = lax.axis_index("s")
    wid = s * NC + c
    _zero_fill(zbuf, f)
    base = s * NSLICE
    pltpu.sync_copy(zbuf, acc_s.at[pl.ds(base, NSLICE)])

    start = wid * RW
    pltpu.sync_copy(src3d.at[pl.ds(start, RW)], sbuf)
    pltpu.sync_copy(dst3d.at[pl.ds(start, RW)], dbuf)
    pltpu.async_copy(hs.at[sbuf.at[0, 0]], rows.at[0], sem0)
    pltpu.async_copy(hs.at[sbuf.at[0, 0]], rows.at[1], sem1)
    pltpu.make_async_copy(hs.at[sbuf.at[0, 0]], rows.at[0], sem0).wait()
    pltpu.make_async_copy(hs.at[sbuf.at[0, 0]], rows.at[1], sem1).wait()
    plsc.subcore_barrier()

    def pair(p, _):
        j0 = 2 * p
        j1 = j0 + 1
        r0 = start + j0
        r1 = start + j1

        @pl.when(r0 < NROWS)
        def _():
            pltpu.async_copy(hs.at[sbuf.at[j0, 0]], rows.at[0], sem0)

        @pl.when(r1 < NROWS)
        def _():
            pltpu.async_copy(hs.at[sbuf.at[j1, 0]], rows.at[1], sem1)

        @pl.when(r0 < NROWS)
        def _():
            pltpu.make_async_copy(hs.at[sbuf.at[j0, 0]], rows.at[0], sem0).wait()
        pltpu.sync_copy(rows.at[0], acc_s.at[dbuf.at[j0, 0]], add=True)

        @pl.when(r1 < NROWS)
        def _():
            pltpu.make_async_copy(hs.at[sbuf.at[j1, 0]], rows.at[1], sem1).wait()
        pltpu.sync_copy(rows.at[1], acc_s.at[dbuf.at[j1, 0]], add=True)
        return 0
    lax.fori_loop(0, RW // 2, pair, 0)
    plsc.subcore_barrier()
    pltpu.sync_copy(acc_s.at[pl.ds(base, NSLICE)], aggp.at[c, pl.ds(base, NSLICE)])


def _make_sc_conv(f):
    return pl.kernel(
        functools.partial(_sc_conv_body, f),
        out_type=pltpu.HBM((NC, N_PAD, f), jnp.float32),
        mesh=_mesh,
        compiler_params=_sc_params,
        scratch_types=[
            pltpu.VMEM((RW, 1, ROW), jnp.int32),
            pltpu.VMEM((RW, 1, ROW), jnp.int32),
            pltpu.VMEM((2, ROW, f), jnp.float32),
            pltpu.VMEM((NSLICE, f), jnp.float32),
            pltpu.VMEM_SHARED((N_PAD, f), jnp.float32),
            pltpu.SemaphoreType.DMA,
            pltpu.SemaphoreType.DMA,
        ],
    )


_sc_conv1 = _make_sc_conv(F1)
_sc_conv2 = _make_sc_conv(F2)


BN = 1000


def _node_body(x_ref, wn_ref, bn_ref, we_ref, be_ref, aggp_ref, degp_ref,
               hs1_ref, norms_ref):
    agg_attr = aggp_ref[0] + aggp_ref[1]
    deg = degp_ref[0] + degp_ref[1]
    in_deg = deg[:, 0:1]
    out_deg = deg[:, 1:2]
    attr_mean = agg_attr / jnp.maximum(in_deg, 1.0)
    e_mean = jnp.dot(attr_mean, we_ref[...],
                     preferred_element_type=jnp.float32) + be_ref[...]
    edge_h = jnp.where(in_deg > 0, jax.nn.relu(e_mean), 0.0)
    node_h = jax.nn.relu(
        jnp.dot(x_ref[...], wn_ref[...], preferred_element_type=jnp.float32)
        + bn_ref[...])
    nsrc = jnp.where(out_deg > 0, lax.rsqrt(out_deg), 0.0)
    ndst = jnp.where(in_deg > 0, lax.rsqrt(in_deg), 0.0)
    h = jnp.concatenate([edge_h, node_h], axis=1)
    hs1_ref[...] = jnp.concatenate(
        [h * nsrc, jnp.zeros((BN, F1 - 24), jnp.float32)], axis=1)
    norms_ref[...] = jnp.concatenate(
        [nsrc, ndst, jnp.zeros((BN, 6), jnp.float32)], axis=1)


def _tc_node(x, Wn, bn, We, be, aggp, degp):
    return pl.pallas_call(
        _node_body,
        grid=(N // BN,),
        in_specs=[
            pl.BlockSpec((BN, D_NODE), lambda i: (i, 0)),
            pl.BlockSpec((D_NODE, NODE_OUT), lambda i: (0, 0)),
            pl.BlockSpec((1, NODE_OUT), lambda i: (0, 0)),
            pl.BlockSpec((D_EDGE, EDGE_OUT), lambda i: (0, 0)),
            pl.BlockSpec((1, EDGE_OUT), lambda i: (0, 0)),
            pl.BlockSpec((NC, BN, 16), lambda i: (0, i, 0)),
            pl.BlockSpec((NC, BN, 16), lambda i: (0, i, 0)),
        ],
        out_specs=[
            pl.BlockSpec((BN, F1), lambda i: (i, 0)),
            pl.BlockSpec((BN, 8), lambda i: (i, 0)),
        ],
        out_shape=[jax.ShapeDtypeStruct((N, F1), jnp.float32),
                   jax.ShapeDtypeStruct((N, 8), jnp.float32)],
    )(x, Wn, bn.reshape(1, NODE_OUT), We, be.reshape(1, EDGE_OUT), aggp, degp)


def _mid_body(aggp_ref, norms_ref, w1_ref, b1_ref, hs2_ref):
    agg = aggp_ref[0] + aggp_ref[1]
    ndst = norms_ref[:, 1:2]
    nsrc = norms_ref[:, 0:1]
    a = agg[:, :24] * ndst
    h1 = jax.nn.relu(
        jnp.dot(a, w1_ref[...], preferred_element_type=jnp.float32)
        + b1_ref[...])
    hs2_ref[...] = jnp.concatenate(
        [h1 * nsrc, jnp.zeros((BN, F2 - HIDDEN), jnp.float32)], axis=1)


def _tc_mid(aggp, norms, W1, b1):
    return pl.pallas_call(
        _mid_body,
        grid=(N // BN,),
        in_specs=[
            pl.BlockSpec((NC, BN, F1), lambda i: (0, i, 0)),
            pl.BlockSpec((BN, 8), lambda i: (i, 0)),
            pl.BlockSpec((24, HIDDEN), lambda i: (0, 0)),
            pl.BlockSpec((1, HIDDEN), lambda i: (0, 0)),
        ],
        out_specs=pl.BlockSpec((BN, F2), lambda i: (i, 0)),
        out_shape=jax.ShapeDtypeStruct((N, F2), jnp.float32),
    )(aggp, norms, W1, b1.reshape(1, HIDDEN))


def _out_body(aggp_ref, norms_ref, w2_ref, b2_ref, out_ref):
    agg = aggp_ref[0] + aggp_ref[1]
    ndst = norms_ref[:, 1:2]
    a = agg[:, :HIDDEN] * ndst
    out_ref[...] = (jnp.dot(a, w2_ref[...], preferred_element_type=jnp.float32)
                    + b2_ref[...])


def _tc_out(aggp, norms, W2, b2):
    return pl.pallas_call(
        _out_body,
        grid=(N // BN,),
        in_specs=[
            pl.BlockSpec((NC, BN, F2), lambda i: (0, i, 0)),
            pl.BlockSpec((BN, 8), lambda i: (i, 0)),
            pl.BlockSpec((HIDDEN, NUM_CLASSES), lambda i: (0, 0)),
            pl.BlockSpec((1, NUM_CLASSES), lambda i: (0, 0)),
        ],
        out_specs=pl.BlockSpec((BN, NUM_CLASSES), lambda i: (i, 0)),
        out_shape=jax.ShapeDtypeStruct((N, NUM_CLASSES), jnp.float32),
    )(aggp, norms, W2, b2.reshape(1, NUM_CLASSES))


def kernel(x, edge_index, edge_attr, We, be, Wn, bn, W1, b1, W2, b2):
    npad = NROWS_PAD * ROW - E
    pad = jnp.full((npad,), PAD_NODE, jnp.int32)
    src3d = jnp.concatenate([edge_index[0], pad]).reshape(NROWS_PAD, 1, ROW)
    dst3d = jnp.concatenate([edge_index[1], pad]).reshape(NROWS_PAD, 1, ROW)
    attr3d = edge_attr.reshape(NROWS, ROW, D_EDGE)

    aggp, degp = _sc_edge(attr3d, src3d, dst3d)
    hs1, norms = _tc_node(x, Wn, bn, We, be, aggp, degp)
    agg1p = _sc_conv1(hs1, src3d, dst3d)
    hs2 = _tc_mid(agg1p, norms, W1, b1)
    agg2p = _sc_conv2(hs2, src3d, dst3d)
    return _tc_out(agg2p, norms, W2, b2)


# speedup vs baseline: 13.1103x; 1.2531x over previous
import functools

import jax
import jax.numpy as jnp
from jax import lax
from jax.experimental import pallas as pl
from jax.experimental.pallas import tpu as pltpu
from jax.experimental.pallas import tpu_sc as plsc

N = 10000
E = 320000
D_NODE = 128
D_EDGE = 16
NODE_OUT = 16
EDGE_OUT = 8
HIDDEN = 10
NUM_CLASSES = 2

NC, NS, L = 2, 16, 16
NW = NC * NS
ROW = 128
NROWS = E // ROW
RW = 80
NROWS_PAD = RW * NW
NSLICE = 632
N_PAD = NSLICE * NS
PAD_NODE = N
NBUF = 4

F1 = 32
F2 = 16

_mesh = plsc.VectorSubcoreMesh(core_axis_name="c", subcore_axis_name="s",
                               num_cores=NC, num_subcores=NS)
_sc_params = pltpu.CompilerParams(use_tc_tiling_on_sc=False)


def _zero_fill(zbuf, f):
    def body(r, _):
        for j in range(f // L):
            zbuf[r, pl.ds(j * L, L)] = jnp.zeros((L,), jnp.float32)
        return 0
    lax.fori_loop(0, zbuf.shape[0], body, 0)


def _worker_id():
    return lax.axis_index("s") * NC + lax.axis_index("c")


def _sc_deg_body(src2d, dst2d, degp,
                 sbuf, dbuf, onesd, oness, zbuf, deg_s, sems):
    s = lax.axis_index("s")
    wid = _worker_id()
    _zero_fill(zbuf, 16)

    def ofill(r, _):
        onesd[r, :] = jnp.where(lax.iota(jnp.int32, L) == 0, 1.0, 0.0)
        oness[r, :] = jnp.where(lax.iota(jnp.int32, L) == 1, 1.0, 0.0)
        return 0
    lax.fori_loop(0, ROW, ofill, 0)

    base = s * NSLICE
    pltpu.sync_copy(zbuf, deg_s.at[pl.ds(base, NSLICE)])
    start = wid * RW
    pltpu.sync_copy(src2d.at[pl.ds(start, RW)], sbuf)
    pltpu.sync_copy(dst2d.at[pl.ds(start, RW)], dbuf)
    plsc.subcore_barrier()

    def body(p, _):
        j0 = 2 * p
        j1 = j0 + 1
        pltpu.async_copy(onesd, deg_s.at[dbuf.at[j0]], sems.at[0], add=True)
        pltpu.async_copy(oness, deg_s.at[sbuf.at[j0]], sems.at[1], add=True)
        pltpu.async_copy(onesd, deg_s.at[dbuf.at[j1]], sems.at[2], add=True)
        pltpu.async_copy(oness, deg_s.at[sbuf.at[j1]], sems.at[3], add=True)
        pltpu.make_async_copy(onesd, deg_s.at[dbuf.at[j0]], sems.at[0]).wait()
        pltpu.make_async_copy(oness, deg_s.at[sbuf.at[j0]], sems.at[1]).wait()
        pltpu.make_async_copy(onesd, deg_s.at[dbuf.at[j1]], sems.at[2]).wait()
        pltpu.make_async_copy(oness, deg_s.at[sbuf.at[j1]], sems.at[3]).wait()
        return 0
    lax.fori_loop(0, RW // 2, body, 0)
    plsc.subcore_barrier()
    pltpu.sync_copy(deg_s.at[pl.ds(base, NSLICE)], degp.at[lax.axis_index("c"), pl.ds(base, NSLICE)])


_sc_deg = pl.kernel(
    _sc_deg_body,
    out_type=pltpu.HBM((NC, N_PAD, 16), jnp.float32),
    mesh=_mesh,
    compiler_params=_sc_params,
    scratch_types=[
        pltpu.VMEM((RW, ROW), jnp.int32),
        pltpu.VMEM((RW, ROW), jnp.int32),
        pltpu.VMEM((ROW, 16), jnp.float32),
        pltpu.VMEM((ROW, 16), jnp.float32),
        pltpu.VMEM((NSLICE, 16), jnp.float32),
        pltpu.VMEM_SHARED((N_PAD, 16), jnp.float32),
        pltpu.SemaphoreType.DMA((NBUF,)),
    ],
)


def _sc_attr_body(attr3d, dst2d, aggp,
                  dbuf, ev, zbuf, agg_s, seml, sems):
    s = lax.axis_index("s")
    wid = _worker_id()
    _zero_fill(zbuf, 16)
    base = s * NSLICE
    pltpu.sync_copy(zbuf, agg_s.at[pl.ds(base, NSLICE)])
    start = wid * RW
    pltpu.sync_copy(dst2d.at[pl.ds(start, RW)], dbuf)
    for b in range(NBUF):
        pltpu.sync_copy(attr3d.at[0], ev.at[b])
    plsc.subcore_barrier()

    def body(p, _):
        js = [NBUF * p + b for b in range(NBUF)]
        for b in range(NBUF):
            r = start + js[b]

            @pl.when(p > 0)
            def _(b=b, jp=js[b] - NBUF):
                pltpu.make_async_copy(ev.at[b], agg_s.at[dbuf.at[jp]],
                                      sems.at[b]).wait()

            @pl.when(r < NROWS)
            def _(b=b, r=r):
                pltpu.async_copy(attr3d.at[r], ev.at[b], seml.at[b])
        for b in range(NBUF):
            r = start + js[b]

            @pl.when(r < NROWS)
            def _(b=b, r=r):
                pltpu.make_async_copy(attr3d.at[r], ev.at[b], seml.at[b]).wait()
            pltpu.async_copy(ev.at[b], agg_s.at[dbuf.at[js[b]]], sems.at[b],
                             add=True)
        return 0
    lax.fori_loop(0, RW // NBUF, body, 0)
    for b in range(NBUF):
        pltpu.make_async_copy(ev.at[b], agg_s.at[dbuf.at[RW - NBUF + b]],
                              sems.at[b]).wait()
    plsc.subcore_barrier()
    pltpu.sync_copy(agg_s.at[pl.ds(base, NSLICE)], aggp.at[lax.axis_index("c"), pl.ds(base, NSLICE)])


_sc_attr = pl.kernel(
    _sc_attr_body,
    out_type=pltpu.HBM((NC, N_PAD, 16), jnp.float32),
    mesh=_mesh,
    compiler_params=_sc_params,
    scratch_types=[
        pltpu.VMEM((RW, ROW), jnp.int32),
        pltpu.VMEM((NBUF, ROW, 16), jnp.float32),
        pltpu.VMEM((NSLICE, 16), jnp.float32),
        pltpu.VMEM_SHARED((N_PAD, 16), jnp.float32),
        pltpu.SemaphoreType.DMA((NBUF,)),
        pltpu.SemaphoreType.DMA((NBUF,)),
    ],
)


def _sc_conv_body(f, hs, src2d, dst2d, aggp,
                  sbuf, dbuf, rows, zbuf, acc_s, semg, sems):
    s = lax.axis_index("s")
    wid = _worker_id()
    _zero_fill(zbuf, f)
    base = s * NSLICE
    pltpu.sync_copy(zbuf, acc_s.at[pl.ds(base, NSLICE)])
    start = wid * RW
    pltpu.sync_copy(src2d.at[pl.ds(start, RW)], sbuf)
    pltpu.sync_copy(dst2d.at[pl.ds(start, RW)], dbuf)
    for b in range(NBUF):
        pltpu.async_copy(hs.at[sbuf.at[0]], rows.at[b], semg.at[b])
    for b in range(NBUF):
        pltpu.make_async_copy(hs.at[sbuf.at[0]], rows.at[b], semg.at[b]).wait()
    plsc.subcore_barrier()

    def body(p, _):
        js = [NBUF * p + b for b in range(NBUF)]
        for b in range(NBUF):
            r = start + js[b]

            @pl.when(p > 0)
            def _(b=b, jp=js[b] - NBUF):
                pltpu.make_async_copy(rows.at[b], acc_s.at[dbuf.at[jp]],
                                      sems.at[b]).wait()

            @pl.when(r < NROWS)
            def _(b=b, j=js[b]):
                pltpu.async_copy(hs.at[sbuf.at[j]], rows.at[b], semg.at[b])
        for b in range(NBUF):
            r = start + js[b]

            @pl.when(r < NROWS)
            def _(b=b, j=js[b]):
                pltpu.make_async_copy(hs.at[sbuf.at[j]], rows.at[b],
                                      semg.at[b]).wait()
            pltpu.async_copy(rows.at[b], acc_s.at[dbuf.at[js[b]]], sems.at[b],
                             add=True)
        return 0
    lax.fori_loop(0, RW // NBUF, body, 0)
    for b in range(NBUF):
        pltpu.make_async_copy(rows.at[b], acc_s.at[dbuf.at[RW - NBUF + b]],
                              sems.at[b]).wait()
    plsc.subcore_barrier()
    pltpu.sync_copy(acc_s.at[pl.ds(base, NSLICE)], aggp.at[lax.axis_index("c"), pl.ds(base, NSLICE)])


def _make_sc_conv(f):
    return pl.kernel(
        functools.partial(_sc_conv_body, f),
        out_type=pltpu.HBM((NC, N_PAD, f), jnp.float32),
        mesh=_mesh,
        compiler_params=_sc_params,
        scratch_types=[
            pltpu.VMEM((RW, ROW), jnp.int32),
            pltpu.VMEM((RW, ROW), jnp.int32),
            pltpu.VMEM((NBUF, ROW, f), jnp.float32),
            pltpu.VMEM((NSLICE, f), jnp.float32),
            pltpu.VMEM_SHARED((N_PAD, f), jnp.float32),
            pltpu.SemaphoreType.DMA((NBUF,)),
            pltpu.SemaphoreType.DMA((NBUF,)),
        ],
    )


_sc_conv1 = _make_sc_conv(F1)
_sc_conv2 = _make_sc_conv(F2)


BN = 1000


def _node_body(x_ref, wn_ref, bn_ref, we_ref, be_ref, aggp_ref, degp_ref,
               hs1_ref, norms_ref):
    agg_attr = aggp_ref[0] + aggp_ref[1]
    deg = degp_ref[0] + degp_ref[1]
    in_deg = deg[:, 0:1]
    out_deg = deg[:, 1:2]
    attr_mean = agg_attr / jnp.maximum(in_deg, 1.0)
    e_mean = jnp.dot(attr_mean, we_ref[...],
                     preferred_element_type=jnp.float32) + be_ref[...]
    edge_h = jnp.where(in_deg > 0, jax.nn.relu(e_mean), 0.0)
    node_h = jax.nn.relu(
        jnp.dot(x_ref[...], wn_ref[...], preferred_element_type=jnp.float32)
        + bn_ref[...])
    nsrc = jnp.where(out_deg > 0, lax.rsqrt(out_deg), 0.0)
    ndst = jnp.where(in_deg > 0, lax.rsqrt(in_deg), 0.0)
    h = jnp.concatenate([edge_h, node_h], axis=1)
    hs1_ref[...] = jnp.concatenate(
        [h * nsrc, jnp.zeros((BN, F1 - 24), jnp.float32)], axis=1)
    norms_ref[...] = jnp.concatenate(
        [nsrc, ndst, jnp.zeros((BN, 6), jnp.float32)], axis=1)


def _tc_node(x, Wn, bn, We, be, aggp, degp):
    return pl.pallas_call(
        _node_body,
        grid=(N // BN,),
        in_specs=[
            pl.BlockSpec((BN, D_NODE), lambda i: (i, 0)),
            pl.BlockSpec((D_NODE, NODE_OUT), lambda i: (0, 0)),
            pl.BlockSpec((1, NODE_OUT), lambda i: (0, 0)),
            pl.BlockSpec((D_EDGE, EDGE_OUT), lambda i: (0, 0)),
            pl.BlockSpec((1, EDGE_OUT), lambda i: (0, 0)),
            pl.BlockSpec((NC, BN, 16), lambda i: (0, i, 0)),
            pl.BlockSpec((NC, BN, 16), lambda i: (0, i, 0)),
        ],
        out_specs=[
            pl.BlockSpec((BN, F1), lambda i: (i, 0)),
            pl.BlockSpec((BN, 8), lambda i: (i, 0)),
        ],
        out_shape=[jax.ShapeDtypeStruct((N, F1), jnp.float32),
                   jax.ShapeDtypeStruct((N, 8), jnp.float32)],
    )(x, Wn, bn.reshape(1, NODE_OUT), We, be.reshape(1, EDGE_OUT), aggp, degp)


def _mid_body(aggp_ref, norms_ref, w1_ref, b1_ref, hs2_ref):
    agg = aggp_ref[0] + aggp_ref[1]
    ndst = norms_ref[:, 1:2]
    nsrc = norms_ref[:, 0:1]
    a = agg[:, :24] * ndst
    h1 = jax.nn.relu(
        jnp.dot(a, w1_ref[...], preferred_element_type=jnp.float32)
        + b1_ref[...])
    hs2_ref[...] = jnp.concatenate(
        [h1 * nsrc, jnp.zeros((BN, F2 - HIDDEN), jnp.float32)], axis=1)


def _tc_mid(aggp, norms, W1, b1):
    return pl.pallas_call(
        _mid_body,
        grid=(N // BN,),
        in_specs=[
            pl.BlockSpec((NC, BN, F1), lambda i: (0, i, 0)),
            pl.BlockSpec((BN, 8), lambda i: (i, 0)),
            pl.BlockSpec((24, HIDDEN), lambda i: (0, 0)),
            pl.BlockSpec((1, HIDDEN), lambda i: (0, 0)),
        ],
        out_specs=pl.BlockSpec((BN, F2), lambda i: (i, 0)),
        out_shape=jax.ShapeDtypeStruct((N, F2), jnp.float32),
    )(aggp, norms, W1, b1.reshape(1, HIDDEN))


def _out_body(aggp_ref, norms_ref, w2_ref, b2_ref, out_ref):
    agg = aggp_ref[0] + aggp_ref[1]
    ndst = norms_ref[:, 1:2]
    a = agg[:, :HIDDEN] * ndst
    out_ref[...] = (jnp.dot(a, w2_ref[...], preferred_element_type=jnp.float32)
                    + b2_ref[...])


def _tc_out(aggp, norms, W2, b2):
    return pl.pallas_call(
        _out_body,
        grid=(N // BN,),
        in_specs=[
            pl.BlockSpec((NC, BN, F2), lambda i: (0, i, 0)),
            pl.BlockSpec((BN, 8), lambda i: (i, 0)),
            pl.BlockSpec((HIDDEN, NUM_CLASSES), lambda i: (0, 0)),
            pl.BlockSpec((1, NUM_CLASSES), lambda i: (0, 0)),
        ],
        out_specs=pl.BlockSpec((BN, NUM_CLASSES), lambda i: (i, 0)),
        out_shape=jax.ShapeDtypeStruct((N, NUM_CLASSES), jnp.float32),
    )(aggp, norms, W2, b2.reshape(1, NUM_CLASSES))


def kernel(x, edge_index, edge_attr, We, be, Wn, bn, W1, b1, W2, b2):
    npad = NROWS_PAD * ROW - E
    pad = jnp.full((npad,), PAD_NODE, jnp.int32)
    src2d = jnp.concatenate([edge_index[0], pad]).reshape(NROWS_PAD, ROW)
    dst2d = jnp.concatenate([edge_index[1], pad]).reshape(NROWS_PAD, ROW)
    attr3d = edge_attr.reshape(NROWS, ROW, D_EDGE)

    degp = _sc_deg(src2d, dst2d)
    aggp = _sc_attr(attr3d, dst2d)
    hs1, norms = _tc_node(x, Wn, bn, We, be, aggp, degp)
    agg1p = _sc_conv1(hs1, src2d, dst2d)
    hs2 = _tc_mid(agg1p, norms, W1, b1)
    agg2p = _sc_conv2(hs2, src2d, dst2d)
    return _tc_out(agg2p, norms, W2, b2)
